# baseline (speedup 1.0000x reference)
import functools

import jax
import jax.numpy as jnp
from jax import lax
from jax.experimental import pallas as pl
from jax.experimental.pallas import tpu as pltpu
from jax.experimental.pallas import tpu_sc as plsc

TT = 256
TILE = 128
LANES = 128


def _router_body(x_ref, wt_ref, b_ref, eid_ref, rank_ref, meta_ref, basex_ref,
                 hist_ref, *, nt, ntiles):
    i = pl.program_id(0)

    @pl.when(i == 0)
    def _init():
        hist_ref[...] = jnp.zeros_like(hist_ref)

    x = x_ref[...]
    logits = jnp.dot(x, wt_ref[...], preferred_element_type=jnp.float32)
    logits = logits + b_ref[0:1, :]
    lane = lax.broadcasted_iota(jnp.int32, logits.shape, 1)
    m = jnp.max(logits, axis=1, keepdims=True)
    cand = jnp.where(logits >= m, lane, LANES)
    eid = jnp.min(cand, axis=1, keepdims=True)
    onehot = (lane == eid).astype(jnp.float32)

    r = lax.broadcasted_iota(jnp.int32, (TT, TT), 0)
    c = lax.broadcasted_iota(jnp.int32, (TT, TT), 1)
    lt = (c < r).astype(jnp.float32)
    prefix = jnp.dot(lt, onehot, preferred_element_type=jnp.float32)
    run = hist_ref[0:1, :]
    rank = (jnp.sum(prefix * onehot, axis=1, keepdims=True)
            + jnp.sum(onehot * run, axis=1, keepdims=True))

    eid_ref[...] = eid
    rank_ref[...] = rank.astype(jnp.int32)
    hist_ref[...] = hist_ref[...] + jnp.sum(onehot, axis=0, keepdims=True)

    @pl.when(i == nt - 1)
    def _meta():
        countf = hist_ref[0:1, :]
        padded = (((countf.astype(jnp.int32) + (TILE - 1)) >> 7) << 7)
        paddedf = padded.astype(jnp.float32)
        rr = lax.broadcasted_iota(jnp.int32, (LANES, LANES), 0)
        cc = lax.broadcasted_iota(jnp.int32, (LANES, LANES), 1)
        ltf = (rr < cc).astype(jnp.float32)
        paddedb = jnp.broadcast_to(paddedf, (8, LANES))
        basef = jnp.dot(paddedb, ltf,
                        preferred_element_type=jnp.float32)[0:1, :]
        validf = (cc[0:1, :] < 8).astype(jnp.float32)
        totalf = jnp.sum(paddedf * validf)
        nu = (totalf * (1.0 / TILE)).astype(jnp.int32)
        base_b = jnp.broadcast_to(basef, (LANES, LANES))
        startf = (rr * TILE).astype(jnp.float32)
        valid_b = cc < 8
        cnt = jnp.sum(jnp.where(jnp.logical_and(base_b <= startf, valid_b),
                                1, 0), axis=1, keepdims=True)
        last = jnp.sum(jnp.where(
            jnp.logical_and(basef <= totalf - TILE, valid_b[0:1, :]), 1, 0)) - 1
        teids = jnp.minimum(cnt - 1, last)
        rows = lax.broadcasted_iota(jnp.int32, (LANES, 1), 0)
        meta_ref[...] = jnp.where(rows == ntiles, nu, teids)
        base_col = jnp.sum(
            jnp.where(jnp.logical_and(rr == cc, valid_b),
                      base_b.astype(jnp.int32), 0),
            axis=1, keepdims=True)
        basex_ref[...] = jnp.broadcast_to(base_col, (LANES, 16))


def _route(tokens, wt_pad, b_pad, ntiles):
    t, h = tokens.shape
    nt = t // TT
    eid, rank, meta, basex = pl.pallas_call(
        functools.partial(_router_body, nt=nt, ntiles=ntiles),
        grid=(nt,),
        in_specs=[
            pl.BlockSpec((TT, h), lambda i: (i, 0)),
            pl.BlockSpec((h, LANES), lambda i: (0, 0)),
            pl.BlockSpec((8, LANES), lambda i: (0, 0)),
        ],
        out_specs=[
            pl.BlockSpec((TT, 1), lambda i: (i, 0)),
            pl.BlockSpec((TT, 1), lambda i: (i, 0)),
            pl.BlockSpec((LANES, 1), lambda i: (0, 0)),
            pl.BlockSpec((LANES, 16), lambda i: (0, 0)),
        ],
        out_shape=[
            jax.ShapeDtypeStruct((t, 1), jnp.int32),
            jax.ShapeDtypeStruct((t, 1), jnp.int32),
            jax.ShapeDtypeStruct((LANES, 1), jnp.int32),
            jax.ShapeDtypeStruct((LANES, 16), jnp.int32),
        ],
        scratch_shapes=[pltpu.VMEM((8, LANES), jnp.float32)],
    )(tokens, wt_pad, b_pad)
    return (eid.reshape(t), rank.reshape(t), meta.reshape(LANES),
            basex.reshape(LANES * 16))


def _dispatch(tokens, rank, eid, basex, n_sorted):
    t, h = tokens.shape
    info = plsc.get_sparse_core_info()
    nw = info.num_cores * info.num_subcores
    cpw = t // nw
    mesh = plsc.VectorSubcoreMesh(core_axis_name="c", subcore_axis_name="s")

    @functools.partial(
        pl.kernel,
        mesh=mesh,
        out_type=[
            jax.ShapeDtypeStruct((n_sorted, h), jnp.float32),
            jax.ShapeDtypeStruct((t,), jnp.int32),
        ],
        scratch_types=[
            pltpu.VMEM((cpw,), jnp.int32),
            pltpu.VMEM((cpw,), jnp.int32),
            pltpu.VMEM((128,), jnp.int32),
            pltpu.VMEM((cpw,), jnp.int32),
            pltpu.VMEM((cpw, h), jnp.float32),
            pltpu.SemaphoreType.DMA,
        ],
    )
    def k(tokens_hbm, rank_hbm, eid_hbm, basex_hbm, xs_hbm, pos_hbm,
          rank_v, eid_v, base_v, pos_v, rows_v, sem):
        wid = lax.axis_index("s") * info.num_cores + lax.axis_index("c")
        off = wid * cpw
        pltpu.sync_copy(rank_hbm.at[pl.ds(off, cpw)], rank_v)
        pltpu.sync_copy(eid_hbm.at[pl.ds(off, cpw)], eid_v)
        pltpu.sync_copy(basex_hbm.at[pl.ds(0, 128)], base_v)
        bs = [base_v[pl.ds(16 * e, 16)] for e in range(8)]
        for j in range(cpw // 16):
            sl = pl.ds(j * 16, 16)
            ev = eid_v[sl]
            bv = bs[7]
            for e in range(6, -1, -1):
                bv = jnp.where(ev == e, bs[e], bv)
            pos_v[sl] = rank_v[sl] + bv
        pltpu.sync_copy(pos_v, pos_hbm.at[pl.ds(off, cpw)])
        pltpu.sync_copy(tokens_hbm.at[pl.ds(off, cpw)], rows_v)
        pltpu.async_copy(rows_v, xs_hbm.at[pos_v], sem).wait()

    return k(tokens, rank, eid, basex)


def _ffn_body(s_ref, x_ref, w1_ref, w2_ref, y_ref, *, ntiles):
    i = pl.program_id(0)

    @pl.when(i < s_ref[ntiles])
    def _():
        x = x_ref[...]
        hmid = lax.dot_general(x, w1_ref[0], (((1,), (1,)), ((), ())),
                               preferred_element_type=jnp.float32)
        hmid = hmid * jax.nn.sigmoid(hmid)
        y_ref[...] = lax.dot_general(hmid, w2_ref[0], (((1,), (1,)), ((), ())),
                                     preferred_element_type=jnp.float32)


def _ffn(sinfo, xs, w1, w2):
    ns, h = xs.shape
    e, f, _ = w1.shape
    ntiles = ns // TILE
    grid_spec = pltpu.PrefetchScalarGridSpec(
        num_scalar_prefetch=1,
        grid=(ntiles,),
        in_specs=[
            pl.BlockSpec((TILE, h), lambda i, s: (i, 0)),
            pl.BlockSpec((1, f, h), lambda i, s: (s[i], 0, 0)),
            pl.BlockSpec((1, h, f), lambda i, s: (s[i], 0, 0)),
        ],
        out_specs=pl.BlockSpec((TILE, h), lambda i, s: (i, 0)),
    )
    return pl.pallas_call(
        functools.partial(_ffn_body, ntiles=ntiles),
        grid_spec=grid_spec,
        out_shape=jax.ShapeDtypeStruct((ns, h), jnp.float32),
    )(sinfo, xs, w1, w2)


def _combine(ys, pos):
    ns, h = ys.shape
    t = pos.shape[0]
    info = plsc.get_sparse_core_info()
    nw = info.num_cores * info.num_subcores
    cpw = t // nw
    mesh = plsc.VectorSubcoreMesh(core_axis_name="c", subcore_axis_name="s")

    @functools.partial(
        pl.kernel,
        mesh=mesh,
        out_type=jax.ShapeDtypeStruct((t, h), jnp.float32),
        scratch_types=[
            pltpu.VMEM((cpw,), jnp.int32),
            pltpu.VMEM((cpw, h), jnp.float32),
            pltpu.SemaphoreType.DMA,
        ],
    )
    def k(ys_hbm, pos_hbm, out_hbm, idx_v, rows_v, sem):
        wid = lax.axis_index("s") * info.num_cores + lax.axis_index("c")
        off = wid * cpw
        pltpu.sync_copy(pos_hbm.at[pl.ds(off, cpw)], idx_v)
        pltpu.async_copy(ys_hbm.at[idx_v], rows_v, sem).wait()
        pltpu.sync_copy(rows_v, out_hbm.at[pl.ds(off, cpw)])

    return k(ys, pos)


def kernel(hidden_states, router_w, router_b, w1, w2):
    b, s, h = hidden_states.shape
    e, f, _ = w1.shape
    tokens = hidden_states.reshape(-1, h)
    t = tokens.shape[0]

    wt_pad = jnp.zeros((h, LANES), jnp.float32).at[:, :e].set(router_w.T)
    b_row = jnp.full((LANES,), -1e30, jnp.float32).at[:e].set(router_b)
    b_pad = jnp.broadcast_to(b_row, (8, LANES))

    n_sorted = t + e * TILE
    ntiles = n_sorted // TILE

    eid, rank, meta, basex = _route(tokens, wt_pad, b_pad, ntiles)
    xs, pos = _dispatch(tokens, rank, eid, basex, n_sorted)
    ys = _ffn(meta, xs, w1, w2)
    out = _combine(ys, pos)
    return out.reshape(b, s, h)


# speedup vs baseline: 1.0890x; 1.0890x over previous
import functools

import jax
import jax.numpy as jnp
from jax import lax
from jax.experimental import pallas as pl
from jax.experimental.pallas import tpu as pltpu
from jax.experimental.pallas import tpu_sc as plsc

TT = 256
TILE = 128
LANES = 128


def _router_body(x_ref, wt_ref, b_ref, code_ref, meta_ref, hist_ref,
                 *, nt, ntiles):
    i = pl.program_id(0)

    @pl.when(i == 0)
    def _init():
        hist_ref[...] = jnp.zeros_like(hist_ref)

    x = x_ref[...]
    logits = jnp.dot(x, wt_ref[...], preferred_element_type=jnp.float32)
    logits = logits + b_ref[0:1, :]
    lane = lax.broadcasted_iota(jnp.int32, logits.shape, 1)
    m = jnp.max(logits, axis=1, keepdims=True)
    cand = jnp.where(logits >= m, lane, LANES)
    eid = jnp.min(cand, axis=1, keepdims=True)
    onehot = (lane == eid).astype(jnp.float32)

    r = lax.broadcasted_iota(jnp.int32, (TT, TT), 0)
    c = lax.broadcasted_iota(jnp.int32, (TT, TT), 1)
    lt = (c < r).astype(jnp.float32)
    prefix = jnp.dot(lt, onehot, preferred_element_type=jnp.float32)
    run = hist_ref[0:1, :]
    rank = (jnp.sum(prefix * onehot, axis=1, keepdims=True)
            + jnp.sum(onehot * run, axis=1, keepdims=True))
    hist_ref[...] = hist_ref[...] + jnp.sum(onehot, axis=0, keepdims=True)

    code = eid.astype(jnp.float32) * 4096.0 + rank
    rr = lax.broadcasted_iota(jnp.int32, (LANES, LANES), 0)
    cc = lax.broadcasted_iota(jnp.int32, (LANES, LANES), 1)
    identf = (rr == cc).astype(jnp.float32)
    for half in range(TT // LANES):
        col = code[half * LANES:(half + 1) * LANES, :]
        row = jnp.sum(identf * col, axis=0, keepdims=True)
        code_ref[pl.ds((TT // LANES) * i + half, 1), :] = row.astype(jnp.int32)

    @pl.when(i == nt - 1)
    def _meta():
        countf = hist_ref[0:1, :]
        padded = (((countf.astype(jnp.int32) + (TILE - 1)) >> 7) << 7)
        paddedf = padded.astype(jnp.float32)
        ltf = (rr < cc).astype(jnp.float32)
        paddedb = jnp.broadcast_to(paddedf, (8, LANES))
        basef = jnp.dot(paddedb, ltf,
                        preferred_element_type=jnp.float32)[0:1, :]
        validf = (cc[0:1, :] < 8).astype(jnp.float32)
        totalf = jnp.sum(paddedf * validf)
        nu = (totalf * (1.0 / TILE)).astype(jnp.int32)
        base_b = jnp.broadcast_to(basef, (LANES, LANES))
        startf = (rr * TILE).astype(jnp.float32)
        valid_b = cc < 8
        cnt = jnp.sum(jnp.where(jnp.logical_and(base_b <= startf, valid_b),
                                1, 0), axis=1, keepdims=True)
        last = jnp.sum(jnp.where(
            jnp.logical_and(basef <= totalf - TILE, valid_b[0:1, :]), 1, 0)) - 1
        teid_col = jnp.minimum(cnt - 1, last).astype(jnp.float32)
        teid_row = jnp.sum(identf * teid_col, axis=0, keepdims=True)
        lane1 = cc[0:1, :]
        sinfo = jnp.where(lane1 == ntiles, nu,
                          teid_row.astype(jnp.int32))
        expandf = (rr == (cc >> 4)).astype(jnp.float32)
        basex = jnp.dot(basef, expandf,
                        preferred_element_type=jnp.float32).astype(jnp.int32)
        clamp = jnp.minimum(lane1, nu - 1)
        rows8 = lax.broadcasted_iota(jnp.int32, (8, LANES), 0)
        meta_ref[...] = jnp.where(
            rows8 == 0, jnp.broadcast_to(sinfo, (8, LANES)),
            jnp.where(rows8 == 1, jnp.broadcast_to(basex, (8, LANES)),
                      jnp.where(rows8 == 2,
                                jnp.broadcast_to(clamp, (8, LANES)), 0)))


def _route(tokens, wt_pad, b_pad, ntiles):
    t, h = tokens.shape
    nt = t // TT
    code, meta = pl.pallas_call(
        functools.partial(_router_body, nt=nt, ntiles=ntiles),
        grid=(nt,),
        in_specs=[
            pl.BlockSpec((TT, h), lambda i: (i, 0)),
            pl.BlockSpec((h, LANES), lambda i: (0, 0)),
            pl.BlockSpec((8, LANES), lambda i: (0, 0)),
        ],
        out_specs=[
            pl.BlockSpec((t // LANES, LANES), lambda i: (0, 0)),
            pl.BlockSpec((8, LANES), lambda i: (0, 0)),
        ],
        out_shape=[
            jax.ShapeDtypeStruct((t // LANES, LANES), jnp.int32),
            jax.ShapeDtypeStruct((8, LANES), jnp.int32),
        ],
        scratch_shapes=[pltpu.VMEM((8, LANES), jnp.float32)],
    )(tokens, wt_pad, b_pad)
    return code.reshape(t), meta.reshape(8 * LANES)


def _dispatch(tokens, code, meta, n_sorted):
    t, h = tokens.shape
    info = plsc.get_sparse_core_info()
    nw = info.num_cores * info.num_subcores
    cpw = t // nw
    mesh = plsc.VectorSubcoreMesh(core_axis_name="c", subcore_axis_name="s")

    @functools.partial(
        pl.kernel,
        mesh=mesh,
        out_type=[
            jax.ShapeDtypeStruct((n_sorted, h), jnp.float32),
            jax.ShapeDtypeStruct((t,), jnp.int32),
        ],
        scratch_types=[
            pltpu.VMEM((cpw,), jnp.int32),
            pltpu.VMEM((128,), jnp.int32),
            pltpu.VMEM((cpw,), jnp.int32),
            pltpu.VMEM((cpw, h), jnp.float32),
            pltpu.SemaphoreType.DMA,
        ],
    )
    def k(tokens_hbm, code_hbm, meta_hbm, xs_hbm, pos_hbm,
          code_v, base_v, pos_v, rows_v, sem):
        wid = lax.axis_index("s") * info.num_cores + lax.axis_index("c")
        off = wid * cpw
        pltpu.sync_copy(code_hbm.at[pl.ds(off, cpw)], code_v)
        pltpu.sync_copy(meta_hbm.at[pl.ds(LANES, LANES)], base_v)
        bs = [base_v[pl.ds(16 * e, 16)] for e in range(8)]
        for j in range(cpw // 16):
            sl = pl.ds(j * 16, 16)
            cv = code_v[sl]
            ev = lax.shift_right_logical(cv, 12)
            rv = jnp.bitwise_and(cv, 4095)
            bv = bs[7]
            for e in range(6, -1, -1):
                bv = jnp.where(ev == e, bs[e], bv)
            pos_v[sl] = rv + bv
        pltpu.sync_copy(pos_v, pos_hbm.at[pl.ds(off, cpw)])
        pltpu.sync_copy(tokens_hbm.at[pl.ds(off, cpw)], rows_v)
        pltpu.async_copy(rows_v, xs_hbm.at[pos_v], sem).wait()

    return k(tokens, code, meta)


def _ffn_body(s_ref, x_ref, w1_ref, w2_ref, y_ref, *, ntiles):
    i = pl.program_id(0)

    @pl.when(i < s_ref[ntiles])
    def _():
        x = x_ref[...]
        hmid = lax.dot_general(x, w1_ref[0], (((1,), (1,)), ((), ())),
                               preferred_element_type=jnp.float32)
        hmid = hmid * jax.nn.sigmoid(hmid)
        y_ref[...] = lax.dot_general(hmid, w2_ref[0], (((1,), (1,)), ((), ())),
                                     preferred_element_type=jnp.float32)


def _ffn(sinfo, xs, w1, w2):
    ns, h = xs.shape
    e, f, _ = w1.shape
    ntiles = ns // TILE
    grid_spec = pltpu.PrefetchScalarGridSpec(
        num_scalar_prefetch=1,
        grid=(ntiles,),
        in_specs=[
            pl.BlockSpec((TILE, h), lambda i, s: (s[2 * LANES + i], 0)),
            pl.BlockSpec((1, f, h), lambda i, s: (s[i], 0, 0)),
            pl.BlockSpec((1, h, f), lambda i, s: (s[i], 0, 0)),
        ],
        out_specs=pl.BlockSpec((TILE, h), lambda i, s: (s[2 * LANES + i], 0)),
    )
    return pl.pallas_call(
        functools.partial(_ffn_body, ntiles=ntiles),
        grid_spec=grid_spec,
        out_shape=jax.ShapeDtypeStruct((ns, h), jnp.float32),
    )(sinfo, xs, w1, w2)


def _combine(ys, pos):
    ns, h = ys.shape
    t = pos.shape[0]
    info = plsc.get_sparse_core_info()
    nw = info.num_cores * info.num_subcores
    cpw = t // nw
    mesh = plsc.VectorSubcoreMesh(core_axis_name="c", subcore_axis_name="s")

    @functools.partial(
        pl.kernel,
        mesh=mesh,
        out_type=jax.ShapeDtypeStruct((t, h), jnp.float32),
        scratch_types=[
            pltpu.VMEM((cpw,), jnp.int32),
            pltpu.VMEM((cpw, h), jnp.float32),
            pltpu.SemaphoreType.DMA,
        ],
    )
    def k(ys_hbm, pos_hbm, out_hbm, idx_v, rows_v, sem):
        wid = lax.axis_index("s") * info.num_cores + lax.axis_index("c")
        off = wid * cpw
        pltpu.sync_copy(pos_hbm.at[pl.ds(off, cpw)], idx_v)
        pltpu.async_copy(ys_hbm.at[idx_v], rows_v, sem).wait()
        pltpu.sync_copy(rows_v, out_hbm.at[pl.ds(off, cpw)])

    return k(ys, pos)


def kernel(hidden_states, router_w, router_b, w1, w2):
    b, s, h = hidden_states.shape
    e, f, _ = w1.shape
    tokens = hidden_states.reshape(-1, h)
    t = tokens.shape[0]

    wt_pad = jnp.zeros((h, LANES), jnp.float32).at[:, :e].set(router_w.T)
    b_row = jnp.full((LANES,), -1e30, jnp.float32).at[:e].set(router_b)
    b_pad = jnp.broadcast_to(b_row, (8, LANES))

    n_sorted = t + e * TILE
    ntiles = n_sorted // TILE

    code, meta = _route(tokens, wt_pad, b_pad, ntiles)
    xs, pos = _dispatch(tokens, code, meta, n_sorted)
    ys = _ffn(meta, xs, w1, w2)
    out = _combine(ys, pos)
    return out.reshape(b, s, h)


# speedup vs baseline: 1.1347x; 1.0420x over previous
import functools

import jax
import jax.numpy as jnp
from jax import lax
from jax.experimental import pallas as pl
from jax.experimental.pallas import tpu as pltpu
from jax.experimental.pallas import tpu_sc as plsc

TT = 256
TILE = 128
LANES = 128


def _router_body(x_ref, wt_ref, b_ref, code_ref, meta_ref, hist_ref,
                 *, nt, ntiles):
    i = pl.program_id(0)

    @pl.when(i == 0)
    def _init():
        hist_ref[...] = jnp.zeros_like(hist_ref)

    x = x_ref[...]
    logits = lax.dot_general(x, wt_ref[...], (((1,), (1,)), ((), ())),
                             preferred_element_type=jnp.float32)
    logits = logits + b_ref[...].reshape(1, 8)
    lane = lax.broadcasted_iota(jnp.int32, logits.shape, 1)
    m = jnp.max(logits, axis=1, keepdims=True)
    cand = jnp.where(logits >= m, lane, 8)
    eid = jnp.min(cand, axis=1, keepdims=True)
    onehot = (lane == eid).astype(jnp.float32)

    r = lax.broadcasted_iota(jnp.int32, (TT, TT), 0)
    c = lax.broadcasted_iota(jnp.int32, (TT, TT), 1)
    lt = (c < r).astype(jnp.float32)
    prefix = jnp.dot(lt, onehot, preferred_element_type=jnp.float32)
    run = hist_ref[0:1, :]
    rank = (jnp.sum(prefix * onehot, axis=1, keepdims=True)
            + jnp.sum(onehot * run, axis=1, keepdims=True))
    hist_ref[...] = hist_ref[...] + jnp.sum(onehot, axis=0, keepdims=True)

    code = eid.astype(jnp.float32) * 4096.0 + rank
    rr = lax.broadcasted_iota(jnp.int32, (LANES, LANES), 0)
    cc = lax.broadcasted_iota(jnp.int32, (LANES, LANES), 1)
    identf = (rr == cc).astype(jnp.float32)
    for half in range(TT // LANES):
        col = code[half * LANES:(half + 1) * LANES, :]
        row = jnp.sum(identf * col, axis=0, keepdims=True)
        code_ref[pl.ds((TT // LANES) * i + half, 1), :] = row.astype(jnp.int32)

    @pl.when(i == nt - 1)
    def _meta():
        countf = hist_ref[0:1, :]
        padded = (((countf.astype(jnp.int32) + (TILE - 1)) >> 7) << 7)
        paddedf = padded.astype(jnp.float32)
        r8 = lax.broadcasted_iota(jnp.int32, (8, 8), 0)
        c8 = lax.broadcasted_iota(jnp.int32, (8, 8), 1)
        lt8 = (r8 < c8).astype(jnp.float32)
        paddedb = jnp.broadcast_to(paddedf, (8, 8))
        basef = jnp.dot(paddedb, lt8,
                        preferred_element_type=jnp.float32)[0:1, :]
        totalf = jnp.sum(paddedf)
        nu = (totalf * (1.0 / TILE)).astype(jnp.int32)
        base_b = jnp.broadcast_to(basef, (LANES, 8))
        startf = (lax.broadcasted_iota(jnp.int32, (LANES, 8), 0)
                  * TILE).astype(jnp.float32)
        cnt = jnp.sum(jnp.where(base_b <= startf, 1, 0),
                      axis=1, keepdims=True)
        last = jnp.sum(jnp.where(basef <= totalf - TILE, 1, 0)) - 1
        teid_col = jnp.minimum(cnt - 1, last).astype(jnp.float32)
        teid_row = jnp.sum(identf * teid_col, axis=0, keepdims=True)
        lane1 = cc[0:1, :]
        sinfo = jnp.where(lane1 == ntiles, nu,
                          teid_row.astype(jnp.int32))
        expandf = (lax.broadcasted_iota(jnp.int32, (8, LANES), 0)
                   == (lax.broadcasted_iota(jnp.int32, (8, LANES), 1) >> 4)
                   ).astype(jnp.float32)
        basex = jnp.dot(basef, expandf,
                        preferred_element_type=jnp.float32).astype(jnp.int32)
        clamp = jnp.minimum(lane1, nu - 1)
        rows8 = lax.broadcasted_iota(jnp.int32, (8, LANES), 0)
        meta_ref[...] = jnp.where(
            rows8 == 0, jnp.broadcast_to(sinfo, (8, LANES)),
            jnp.where(rows8 == 1, jnp.broadcast_to(basex, (8, LANES)),
                      jnp.where(rows8 == 2,
                                jnp.broadcast_to(clamp, (8, LANES)), 0)))


def _route(tokens, router_w, router_b, ntiles):
    t, h = tokens.shape
    nt = t // TT
    code, meta = pl.pallas_call(
        functools.partial(_router_body, nt=nt, ntiles=ntiles),
        grid=(nt,),
        in_specs=[
            pl.BlockSpec((TT, h), lambda i: (i, 0)),
            pl.BlockSpec((8, h), lambda i: (0, 0)),
            pl.BlockSpec((8,), lambda i: (0,)),
        ],
        out_specs=[
            pl.BlockSpec((t // LANES, LANES), lambda i: (0, 0)),
            pl.BlockSpec((8, LANES), lambda i: (0, 0)),
        ],
        out_shape=[
            jax.ShapeDtypeStruct((t // LANES, LANES), jnp.int32),
            jax.ShapeDtypeStruct((8, LANES), jnp.int32),
        ],
        scratch_shapes=[pltpu.VMEM((8, 8), jnp.float32)],
    )(tokens, router_w, router_b)
    return code.reshape(t), meta.reshape(8 * LANES)


def _dispatch(tokens, code, meta, n_sorted):
    t, h = tokens.shape
    info = plsc.get_sparse_core_info()
    nw = info.num_cores * info.num_subcores
    cpw = t // nw
    mesh = plsc.VectorSubcoreMesh(core_axis_name="c", subcore_axis_name="s")

    @functools.partial(
        pl.kernel,
        mesh=mesh,
        out_type=[
            jax.ShapeDtypeStruct((n_sorted, h), jnp.float32),
            jax.ShapeDtypeStruct((t,), jnp.int32),
        ],
        scratch_types=[
            pltpu.VMEM((cpw,), jnp.int32),
            pltpu.VMEM((128,), jnp.int32),
            pltpu.VMEM((cpw,), jnp.int32),
            pltpu.VMEM((cpw, h), jnp.float32),
            pltpu.SemaphoreType.DMA,
        ],
    )
    def k(tokens_hbm, code_hbm, meta_hbm, xs_hbm, pos_hbm,
          code_v, base_v, pos_v, rows_v, sem):
        wid = lax.axis_index("s") * info.num_cores + lax.axis_index("c")
        off = wid * cpw
        pltpu.sync_copy(code_hbm.at[pl.ds(off, cpw)], code_v)
        pltpu.sync_copy(meta_hbm.at[pl.ds(LANES, LANES)], base_v)
        bs = [base_v[pl.ds(16 * e, 16)] for e in range(8)]
        for j in range(cpw // 16):
            sl = pl.ds(j * 16, 16)
            cv = code_v[sl]
            ev = lax.shift_right_logical(cv, 12)
            rv = jnp.bitwise_and(cv, 4095)
            bv = bs[7]
            for e in range(6, -1, -1):
                bv = jnp.where(ev == e, bs[e], bv)
            pos_v[sl] = rv + bv
        pltpu.sync_copy(pos_v, pos_hbm.at[pl.ds(off, cpw)])
        pltpu.sync_copy(tokens_hbm.at[pl.ds(off, cpw)], rows_v)
        pltpu.async_copy(rows_v, xs_hbm.at[pos_v], sem).wait()

    return k(tokens, code, meta)


def _ffn_body(s_ref, x_ref, w1_ref, w2_ref, y_ref, *, ntiles):
    i = pl.program_id(0)

    @pl.when(i < s_ref[ntiles])
    def _():
        x = x_ref[...]
        hmid = lax.dot_general(x, w1_ref[0], (((1,), (1,)), ((), ())),
                               preferred_element_type=jnp.float32)
        hmid = hmid * jax.nn.sigmoid(hmid)
        y_ref[...] = lax.dot_general(hmid, w2_ref[0], (((1,), (1,)), ((), ())),
                                     preferred_element_type=jnp.float32)


def _ffn(sinfo, xs, w1, w2):
    ns, h = xs.shape
    e, f, _ = w1.shape
    ntiles = ns // TILE
    grid_spec = pltpu.PrefetchScalarGridSpec(
        num_scalar_prefetch=1,
        grid=(ntiles,),
        in_specs=[
            pl.BlockSpec((TILE, h), lambda i, s: (s[2 * LANES + i], 0)),
            pl.BlockSpec((1, f, h), lambda i, s: (s[i], 0, 0)),
            pl.BlockSpec((1, h, f), lambda i, s: (s[i], 0, 0)),
        ],
        out_specs=pl.BlockSpec((TILE, h), lambda i, s: (s[2 * LANES + i], 0)),
    )
    return pl.pallas_call(
        functools.partial(_ffn_body, ntiles=ntiles),
        grid_spec=grid_spec,
        out_shape=jax.ShapeDtypeStruct((ns, h), jnp.float32),
    )(sinfo, xs, w1, w2)


def _combine(ys, pos):
    ns, h = ys.shape
    t = pos.shape[0]
    info = plsc.get_sparse_core_info()
    nw = info.num_cores * info.num_subcores
    cpw = t // nw
    mesh = plsc.VectorSubcoreMesh(core_axis_name="c", subcore_axis_name="s")

    @functools.partial(
        pl.kernel,
        mesh=mesh,
        out_type=jax.ShapeDtypeStruct((t, h), jnp.float32),
        scratch_types=[
            pltpu.VMEM((cpw,), jnp.int32),
            pltpu.VMEM((cpw, h), jnp.float32),
            pltpu.SemaphoreType.DMA,
        ],
    )
    def k(ys_hbm, pos_hbm, out_hbm, idx_v, rows_v, sem):
        wid = lax.axis_index("s") * info.num_cores + lax.axis_index("c")
        off = wid * cpw
        pltpu.sync_copy(pos_hbm.at[pl.ds(off, cpw)], idx_v)
        pltpu.async_copy(ys_hbm.at[idx_v], rows_v, sem).wait()
        pltpu.sync_copy(rows_v, out_hbm.at[pl.ds(off, cpw)])

    return k(ys, pos)


def kernel(hidden_states, router_w, router_b, w1, w2):
    b, s, h = hidden_states.shape
    e, f, _ = w1.shape
    tokens = hidden_states.reshape(-1, h)
    t = tokens.shape[0]

    n_sorted = t + e * TILE
    ntiles = n_sorted // TILE

    code, meta = _route(tokens, router_w, router_b, ntiles)
    xs, pos = _dispatch(tokens, code, meta, n_sorted)
    ys = _ffn(meta, xs, w1, w2)
    out = _combine(ys, pos)
    return out.reshape(b, s, h)


# speedup vs baseline: 1.1354x; 1.0006x over previous
import functools

import jax
import jax.numpy as jnp
from jax import lax
from jax.experimental import pallas as pl
from jax.experimental.pallas import tpu as pltpu
from jax.experimental.pallas import tpu_sc as plsc

TT = 256
TILE = 128
LANES = 128


def _router_body(x_ref, wt_ref, b_ref, code_ref, meta_ref, hist_ref,
                 *, nt, ntiles):
    i = pl.program_id(0)

    @pl.when(i == 0)
    def _init():
        hist_ref[...] = jnp.zeros_like(hist_ref)

    x = x_ref[...]
    logits = lax.dot_general(x, wt_ref[...], (((1,), (1,)), ((), ())),
                             preferred_element_type=jnp.float32)
    logits = logits + b_ref[...].reshape(1, 8)
    lane = lax.broadcasted_iota(jnp.int32, logits.shape, 1)
    m = jnp.max(logits, axis=1, keepdims=True)
    cand = jnp.where(logits >= m, lane, 8)
    eid = jnp.min(cand, axis=1, keepdims=True)
    onehot = (lane == eid).astype(jnp.float32)

    r = lax.broadcasted_iota(jnp.int32, (TT, TT), 0)
    c = lax.broadcasted_iota(jnp.int32, (TT, TT), 1)
    lt = (c < r).astype(jnp.float32)
    prefix = jnp.dot(lt, onehot, preferred_element_type=jnp.float32)
    run = hist_ref[0:1, :]
    rank = (jnp.sum(prefix * onehot, axis=1, keepdims=True)
            + jnp.sum(onehot * run, axis=1, keepdims=True))
    hist_ref[...] = hist_ref[...] + jnp.sum(onehot, axis=0, keepdims=True)

    code = eid.astype(jnp.float32) * 4096.0 + rank
    rr = lax.broadcasted_iota(jnp.int32, (LANES, LANES), 0)
    cc = lax.broadcasted_iota(jnp.int32, (LANES, LANES), 1)
    identf = (rr == cc).astype(jnp.float32)
    for half in range(TT // LANES):
        col = code[half * LANES:(half + 1) * LANES, :]
        row = jnp.sum(identf * col, axis=0, keepdims=True)
        code_ref[pl.ds((TT // LANES) * i + half, 1), :] = row.astype(jnp.int32)

    @pl.when(i == nt - 1)
    def _meta():
        countf = hist_ref[0:1, :]
        padded = (((countf.astype(jnp.int32) + (TILE - 1)) >> 7) << 7)
        paddedf = padded.astype(jnp.float32)
        r8 = lax.broadcasted_iota(jnp.int32, (8, 8), 0)
        c8 = lax.broadcasted_iota(jnp.int32, (8, 8), 1)
        lt8 = (r8 < c8).astype(jnp.float32)
        paddedb = jnp.broadcast_to(paddedf, (8, 8))
        basef = jnp.dot(paddedb, lt8,
                        preferred_element_type=jnp.float32)[0:1, :]
        totalf = jnp.sum(paddedf)
        nu = (totalf * (1.0 / TILE)).astype(jnp.int32)
        base_b = jnp.broadcast_to(basef, (LANES, 8))
        startf = (lax.broadcasted_iota(jnp.int32, (LANES, 8), 0)
                  * TILE).astype(jnp.float32)
        cnt = jnp.sum(jnp.where(base_b <= startf, 1, 0),
                      axis=1, keepdims=True)
        last = jnp.sum(jnp.where(basef <= totalf - TILE, 1, 0)) - 1
        teid_col = jnp.minimum(cnt - 1, last).astype(jnp.float32)
        teid_row = jnp.sum(identf * teid_col, axis=0, keepdims=True)
        lane1 = cc[0:1, :]
        sinfo = jnp.where(lane1 == ntiles, nu,
                          teid_row.astype(jnp.int32))
        expandf = (lax.broadcasted_iota(jnp.int32, (8, LANES), 0)
                   == (lax.broadcasted_iota(jnp.int32, (8, LANES), 1) >> 4)
                   ).astype(jnp.float32)
        basex = jnp.dot(basef, expandf,
                        preferred_element_type=jnp.float32).astype(jnp.int32)
        clamp = jnp.minimum(lane1, nu - 1)
        rows8 = lax.broadcasted_iota(jnp.int32, (8, LANES), 0)
        meta_ref[...] = jnp.where(
            rows8 == 0, jnp.broadcast_to(sinfo, (8, LANES)),
            jnp.where(rows8 == 1, jnp.broadcast_to(basex, (8, LANES)),
                      jnp.where(rows8 == 2,
                                jnp.broadcast_to(clamp, (8, LANES)), 0)))


def _route(tokens, router_w, router_b, ntiles):
    t, h = tokens.shape
    nt = t // TT
    code, meta = pl.pallas_call(
        functools.partial(_router_body, nt=nt, ntiles=ntiles),
        grid=(nt,),
        in_specs=[
            pl.BlockSpec((TT, h), lambda i: (i, 0)),
            pl.BlockSpec((8, h), lambda i: (0, 0)),
            pl.BlockSpec((8,), lambda i: (0,)),
        ],
        out_specs=[
            pl.BlockSpec((t // LANES, LANES), lambda i: (0, 0)),
            pl.BlockSpec((8, LANES), lambda i: (0, 0)),
        ],
        out_shape=[
            jax.ShapeDtypeStruct((t // LANES, LANES), jnp.int32),
            jax.ShapeDtypeStruct((8, LANES), jnp.int32),
        ],
        scratch_shapes=[pltpu.VMEM((8, 8), jnp.float32)],
    )(tokens, router_w, router_b)
    return code.reshape(t), meta.reshape(8 * LANES)


def _dispatch(tokens, code, meta, n_sorted):
    t, h = tokens.shape
    info = plsc.get_sparse_core_info()
    nw = info.num_cores * info.num_subcores
    cpw = t // nw
    mesh = plsc.VectorSubcoreMesh(core_axis_name="c", subcore_axis_name="s")

    hc = cpw // 2

    @functools.partial(
        pl.kernel,
        mesh=mesh,
        out_type=[
            jax.ShapeDtypeStruct((n_sorted, h), jnp.float32),
            jax.ShapeDtypeStruct((t,), jnp.int32),
        ],
        scratch_types=[
            pltpu.VMEM((cpw,), jnp.int32),
            pltpu.VMEM((128,), jnp.int32),
            pltpu.VMEM((hc,), jnp.int32),
            pltpu.VMEM((hc,), jnp.int32),
            pltpu.VMEM((hc, h), jnp.float32),
            pltpu.VMEM((hc, h), jnp.float32),
            pltpu.SemaphoreType.DMA,
            pltpu.SemaphoreType.DMA,
            pltpu.SemaphoreType.DMA,
        ],
    )
    def k(tokens_hbm, code_hbm, meta_hbm, xs_hbm, pos_hbm,
          code_v, base_v, pos_v0, pos_v1, rows_v0, rows_v1,
          sem0, sem1, sem2):
        wid = lax.axis_index("s") * info.num_cores + lax.axis_index("c")
        off = wid * cpw
        in0 = pltpu.async_copy(tokens_hbm.at[pl.ds(off, hc)], rows_v0, sem0)
        in1 = pltpu.async_copy(tokens_hbm.at[pl.ds(off + hc, hc)], rows_v1,
                               sem1)
        pltpu.sync_copy(code_hbm.at[pl.ds(off, cpw)], code_v)
        pltpu.sync_copy(meta_hbm.at[pl.ds(LANES, LANES)], base_v)
        bs = [base_v[pl.ds(16 * e, 16)] for e in range(8)]
        for half, pos_v in ((0, pos_v0), (1, pos_v1)):
            for j in range(hc // 16):
                sl16 = pl.ds(half * hc + j * 16, 16)
                cv = code_v[sl16]
                ev = lax.shift_right_logical(cv, 12)
                rv = jnp.bitwise_and(cv, 4095)
                bv = bs[7]
                for e in range(6, -1, -1):
                    bv = jnp.where(ev == e, bs[e], bv)
                pos_v[pl.ds(j * 16, 16)] = rv + bv
        pltpu.sync_copy(pos_v0, pos_hbm.at[pl.ds(off, hc)])
        pltpu.sync_copy(pos_v1, pos_hbm.at[pl.ds(off + hc, hc)])
        in0.wait()
        out0 = pltpu.async_copy(rows_v0, xs_hbm.at[pos_v0], sem2)
        in1.wait()
        out0.wait()
        pltpu.async_copy(rows_v1, xs_hbm.at[pos_v1], sem2).wait()

    return k(tokens, code, meta)


def _ffn_body(s_ref, x_ref, w1_ref, w2_ref, y_ref, *, ntiles):
    i = pl.program_id(0)

    @pl.when(i < s_ref[ntiles])
    def _():
        x = x_ref[...]
        hmid = lax.dot_general(x, w1_ref[0], (((1,), (1,)), ((), ())),
                               preferred_element_type=jnp.float32)
        hmid = hmid * jax.nn.sigmoid(hmid)
        y_ref[...] = lax.dot_general(hmid, w2_ref[0], (((1,), (1,)), ((), ())),
                                     preferred_element_type=jnp.float32)


def _ffn(sinfo, xs, w1, w2):
    ns, h = xs.shape
    e, f, _ = w1.shape
    ntiles = ns // TILE
    grid_spec = pltpu.PrefetchScalarGridSpec(
        num_scalar_prefetch=1,
        grid=(ntiles,),
        in_specs=[
            pl.BlockSpec((TILE, h), lambda i, s: (s[2 * LANES + i], 0)),
            pl.BlockSpec((1, f, h), lambda i, s: (s[i], 0, 0)),
            pl.BlockSpec((1, h, f), lambda i, s: (s[i], 0, 0)),
        ],
        out_specs=pl.BlockSpec((TILE, h), lambda i, s: (s[2 * LANES + i], 0)),
    )
    return pl.pallas_call(
        functools.partial(_ffn_body, ntiles=ntiles),
        grid_spec=grid_spec,
        out_shape=jax.ShapeDtypeStruct((ns, h), jnp.float32),
    )(sinfo, xs, w1, w2)


def _combine(ys, pos):
    ns, h = ys.shape
    t = pos.shape[0]
    info = plsc.get_sparse_core_info()
    nw = info.num_cores * info.num_subcores
    cpw = t // nw
    mesh = plsc.VectorSubcoreMesh(core_axis_name="c", subcore_axis_name="s")

    hc = cpw // 2

    @functools.partial(
        pl.kernel,
        mesh=mesh,
        out_type=jax.ShapeDtypeStruct((t, h), jnp.float32),
        scratch_types=[
            pltpu.VMEM((hc,), jnp.int32),
            pltpu.VMEM((hc,), jnp.int32),
            pltpu.VMEM((hc, h), jnp.float32),
            pltpu.VMEM((hc, h), jnp.float32),
            pltpu.SemaphoreType.DMA,
            pltpu.SemaphoreType.DMA,
            pltpu.SemaphoreType.DMA,
        ],
    )
    def k(ys_hbm, pos_hbm, out_hbm, idx_v0, idx_v1, rows_v0, rows_v1,
          sem0, sem1, sem2):
        wid = lax.axis_index("s") * info.num_cores + lax.axis_index("c")
        off = wid * cpw
        pltpu.sync_copy(pos_hbm.at[pl.ds(off, hc)], idx_v0)
        pltpu.sync_copy(pos_hbm.at[pl.ds(off + hc, hc)], idx_v1)
        g0 = pltpu.async_copy(ys_hbm.at[idx_v0], rows_v0, sem0)
        g1 = pltpu.async_copy(ys_hbm.at[idx_v1], rows_v1, sem1)
        g0.wait()
        w0 = pltpu.async_copy(rows_v0, out_hbm.at[pl.ds(off, hc)], sem2)
        g1.wait()
        w0.wait()
        pltpu.sync_copy(rows_v1, out_hbm.at[pl.ds(off + hc, hc)])

    return k(ys, pos)


def kernel(hidden_states, router_w, router_b, w1, w2):
    b, s, h = hidden_states.shape
    e, f, _ = w1.shape
    tokens = hidden_states.reshape(-1, h)
    t = tokens.shape[0]

    n_sorted = t + e * TILE
    ntiles = n_sorted // TILE

    code, meta = _route(tokens, router_w, router_b, ntiles)
    xs, pos = _dispatch(tokens, code, meta, n_sorted)
    ys = _ffn(meta, xs, w1, w2)
    out = _combine(ys, pos)
    return out.reshape(b, s, h)


# speedup vs baseline: 1.1642x; 1.0254x over previous
import functools

import jax
import jax.numpy as jnp
from jax import lax
from jax.experimental import pallas as pl
from jax.experimental.pallas import tpu as pltpu
from jax.experimental.pallas import tpu_sc as plsc

TT = 512
TILE = 128
LANES = 128


def _router_body(x_ref, wt_ref, b_ref, code_ref, meta_ref, hist_ref,
                 *, nt, ntiles):
    i = pl.program_id(0)

    @pl.when(i == 0)
    def _init():
        hist_ref[...] = jnp.zeros_like(hist_ref)

    x = x_ref[...]
    logits = lax.dot_general(x, wt_ref[...], (((1,), (1,)), ((), ())),
                             preferred_element_type=jnp.float32)
    logits = logits + b_ref[...].reshape(1, 8)
    lane = lax.broadcasted_iota(jnp.int32, logits.shape, 1)
    m = jnp.max(logits, axis=1, keepdims=True)
    cand = jnp.where(logits >= m, lane, 8)
    eid = jnp.min(cand, axis=1, keepdims=True)
    onehot = (lane == eid).astype(jnp.float32)

    r = lax.broadcasted_iota(jnp.int32, (TT, TT), 0)
    c = lax.broadcasted_iota(jnp.int32, (TT, TT), 1)
    lt = (c < r).astype(jnp.float32)
    prefix = jnp.dot(lt, onehot, preferred_element_type=jnp.float32)
    run = hist_ref[0:1, :]
    rank = (jnp.sum(prefix * onehot, axis=1, keepdims=True)
            + jnp.sum(onehot * run, axis=1, keepdims=True))
    hist_ref[...] = hist_ref[...] + jnp.sum(onehot, axis=0, keepdims=True)

    code = eid.astype(jnp.float32) * 4096.0 + rank
    rr = lax.broadcasted_iota(jnp.int32, (LANES, LANES), 0)
    cc = lax.broadcasted_iota(jnp.int32, (LANES, LANES), 1)
    identf = (rr == cc).astype(jnp.float32)
    for half in range(TT // LANES):
        col = code[half * LANES:(half + 1) * LANES, :]
        row = jnp.sum(identf * col, axis=0, keepdims=True)
        code_ref[pl.ds((TT // LANES) * i + half, 1), :] = row.astype(jnp.int32)

    @pl.when(i == nt - 1)
    def _meta():
        countf = hist_ref[0:1, :]
        padded = (((countf.astype(jnp.int32) + (TILE - 1)) >> 7) << 7)
        paddedf = padded.astype(jnp.float32)
        r8 = lax.broadcasted_iota(jnp.int32, (8, 8), 0)
        c8 = lax.broadcasted_iota(jnp.int32, (8, 8), 1)
        lt8 = (r8 < c8).astype(jnp.float32)
        paddedb = jnp.broadcast_to(paddedf, (8, 8))
        basef = jnp.dot(paddedb, lt8,
                        preferred_element_type=jnp.float32)[0:1, :]
        totalf = jnp.sum(paddedf)
        nu = (totalf * (1.0 / TILE)).astype(jnp.int32)
        base_b = jnp.broadcast_to(basef, (LANES, 8))
        startf = (lax.broadcasted_iota(jnp.int32, (LANES, 8), 0)
                  * TILE).astype(jnp.float32)
        cnt = jnp.sum(jnp.where(base_b <= startf, 1, 0),
                      axis=1, keepdims=True)
        last = jnp.sum(jnp.where(basef <= totalf - TILE, 1, 0)) - 1
        teid_col = jnp.minimum(cnt - 1, last).astype(jnp.float32)
        teid_row = jnp.sum(identf * teid_col, axis=0, keepdims=True)
        lane1 = cc[0:1, :]
        sinfo = jnp.where(lane1 == ntiles, nu,
                          teid_row.astype(jnp.int32))
        expandf = (lax.broadcasted_iota(jnp.int32, (8, LANES), 0)
                   == (lax.broadcasted_iota(jnp.int32, (8, LANES), 1) >> 4)
                   ).astype(jnp.float32)
        basex = jnp.dot(basef, expandf,
                        preferred_element_type=jnp.float32).astype(jnp.int32)
        clamp = jnp.minimum(lane1, nu - 1)
        rows8 = lax.broadcasted_iota(jnp.int32, (8, LANES), 0)
        meta_ref[...] = jnp.where(
            rows8 == 0, jnp.broadcast_to(sinfo, (8, LANES)),
            jnp.where(rows8 == 1, jnp.broadcast_to(basex, (8, LANES)),
                      jnp.where(rows8 == 2,
                                jnp.broadcast_to(clamp, (8, LANES)), 0)))


def _route(tokens, router_w, router_b, ntiles):
    t, h = tokens.shape
    nt = t // TT
    code, meta = pl.pallas_call(
        functools.partial(_router_body, nt=nt, ntiles=ntiles),
        grid=(nt,),
        in_specs=[
            pl.BlockSpec((TT, h), lambda i: (i, 0)),
            pl.BlockSpec((8, h), lambda i: (0, 0)),
            pl.BlockSpec((8,), lambda i: (0,)),
        ],
        out_specs=[
            pl.BlockSpec((t // LANES, LANES), lambda i: (0, 0)),
            pl.BlockSpec((8, LANES), lambda i: (0, 0)),
        ],
        out_shape=[
            jax.ShapeDtypeStruct((t // LANES, LANES), jnp.int32),
            jax.ShapeDtypeStruct((8, LANES), jnp.int32),
        ],
        scratch_shapes=[pltpu.VMEM((8, 8), jnp.float32)],
    )(tokens, router_w, router_b)
    return code.reshape(t), meta.reshape(8 * LANES)


def _dispatch(tokens, code, meta, n_sorted):
    t, h = tokens.shape
    info = plsc.get_sparse_core_info()
    nw = info.num_cores * info.num_subcores
    cpw = t // nw
    mesh = plsc.VectorSubcoreMesh(core_axis_name="c", subcore_axis_name="s")

    hc = cpw // 2

    @functools.partial(
        pl.kernel,
        mesh=mesh,
        out_type=[
            jax.ShapeDtypeStruct((n_sorted, h), jnp.float32),
            jax.ShapeDtypeStruct((t,), jnp.int32),
        ],
        scratch_types=[
            pltpu.VMEM((cpw,), jnp.int32),
            pltpu.VMEM((128,), jnp.int32),
            pltpu.VMEM((hc,), jnp.int32),
            pltpu.VMEM((hc,), jnp.int32),
            pltpu.VMEM((hc, h), jnp.float32),
            pltpu.VMEM((hc, h), jnp.float32),
            pltpu.SemaphoreType.DMA,
            pltpu.SemaphoreType.DMA,
            pltpu.SemaphoreType.DMA,
        ],
    )
    def k(tokens_hbm, code_hbm, meta_hbm, xs_hbm, pos_hbm,
          code_v, base_v, pos_v0, pos_v1, rows_v0, rows_v1,
          sem0, sem1, sem2):
        wid = lax.axis_index("s") * info.num_cores + lax.axis_index("c")
        off = wid * cpw
        in0 = pltpu.async_copy(tokens_hbm.at[pl.ds(off, hc)], rows_v0, sem0)
        in1 = pltpu.async_copy(tokens_hbm.at[pl.ds(off + hc, hc)], rows_v1,
                               sem1)
        pltpu.sync_copy(code_hbm.at[pl.ds(off, cpw)], code_v)
        pltpu.sync_copy(meta_hbm.at[pl.ds(LANES, LANES)], base_v)
        bs = [base_v[pl.ds(16 * e, 16)] for e in range(8)]
        for half, pos_v in ((0, pos_v0), (1, pos_v1)):
            for j in range(hc // 16):
                sl16 = pl.ds(half * hc + j * 16, 16)
                cv = code_v[sl16]
                ev = lax.shift_right_logical(cv, 12)
                rv = jnp.bitwise_and(cv, 4095)
                bv = bs[7]
                for e in range(6, -1, -1):
                    bv = jnp.where(ev == e, bs[e], bv)
                pos_v[pl.ds(j * 16, 16)] = rv + bv
        pltpu.sync_copy(pos_v0, pos_hbm.at[pl.ds(off, hc)])
        pltpu.sync_copy(pos_v1, pos_hbm.at[pl.ds(off + hc, hc)])
        in0.wait()
        out0 = pltpu.async_copy(rows_v0, xs_hbm.at[pos_v0], sem2)
        in1.wait()
        out0.wait()
        pltpu.async_copy(rows_v1, xs_hbm.at[pos_v1], sem2).wait()

    return k(tokens, code, meta)


def _ffn_body(s_ref, x_ref, w1_ref, w2_ref, y_ref, *, ntiles):
    i = pl.program_id(0)

    @pl.when(i < s_ref[ntiles])
    def _():
        x = x_ref[...]
        hmid = lax.dot_general(x, w1_ref[0], (((1,), (1,)), ((), ())),
                               preferred_element_type=jnp.float32)
        hmid = hmid * jax.nn.sigmoid(hmid)
        y_ref[...] = lax.dot_general(hmid, w2_ref[0], (((1,), (1,)), ((), ())),
                                     preferred_element_type=jnp.float32)


def _ffn(sinfo, xs, w1, w2):
    ns, h = xs.shape
    e, f, _ = w1.shape
    ntiles = ns // TILE
    grid_spec = pltpu.PrefetchScalarGridSpec(
        num_scalar_prefetch=1,
        grid=(ntiles,),
        in_specs=[
            pl.BlockSpec((TILE, h), lambda i, s: (s[2 * LANES + i], 0)),
            pl.BlockSpec((1, f, h), lambda i, s: (s[i], 0, 0)),
            pl.BlockSpec((1, h, f), lambda i, s: (s[i], 0, 0)),
        ],
        out_specs=pl.BlockSpec((TILE, h), lambda i, s: (s[2 * LANES + i], 0)),
    )
    return pl.pallas_call(
        functools.partial(_ffn_body, ntiles=ntiles),
        grid_spec=grid_spec,
        out_shape=jax.ShapeDtypeStruct((ns, h), jnp.float32),
    )(sinfo, xs, w1, w2)


def _combine(ys, pos):
    ns, h = ys.shape
    t = pos.shape[0]
    info = plsc.get_sparse_core_info()
    nw = info.num_cores * info.num_subcores
    cpw = t // nw
    mesh = plsc.VectorSubcoreMesh(core_axis_name="c", subcore_axis_name="s")

    hc = cpw // 2

    @functools.partial(
        pl.kernel,
        mesh=mesh,
        out_type=jax.ShapeDtypeStruct((t, h), jnp.float32),
        scratch_types=[
            pltpu.VMEM((hc,), jnp.int32),
            pltpu.VMEM((hc,), jnp.int32),
            pltpu.VMEM((hc, h), jnp.float32),
            pltpu.VMEM((hc, h), jnp.float32),
            pltpu.SemaphoreType.DMA,
            pltpu.SemaphoreType.DMA,
            pltpu.SemaphoreType.DMA,
        ],
    )
    def k(ys_hbm, pos_hbm, out_hbm, idx_v0, idx_v1, rows_v0, rows_v1,
          sem0, sem1, sem2):
        wid = lax.axis_index("s") * info.num_cores + lax.axis_index("c")
        off = wid * cpw
        pltpu.sync_copy(pos_hbm.at[pl.ds(off, hc)], idx_v0)
        pltpu.sync_copy(pos_hbm.at[pl.ds(off + hc, hc)], idx_v1)
        g0 = pltpu.async_copy(ys_hbm.at[idx_v0], rows_v0, sem0)
        g1 = pltpu.async_copy(ys_hbm.at[idx_v1], rows_v1, sem1)
        g0.wait()
        w0 = pltpu.async_copy(rows_v0, out_hbm.at[pl.ds(off, hc)], sem2)
        g1.wait()
        w0.wait()
        pltpu.sync_copy(rows_v1, out_hbm.at[pl.ds(off + hc, hc)])

    return k(ys, pos)


def kernel(hidden_states, router_w, router_b, w1, w2):
    b, s, h = hidden_states.shape
    e, f, _ = w1.shape
    tokens = hidden_states.reshape(-1, h)
    t = tokens.shape[0]

    n_sorted = t + e * TILE
    ntiles = n_sorted // TILE

    code, meta = _route(tokens, router_w, router_b, ntiles)
    xs, pos = _dispatch(tokens, code, meta, n_sorted)
    ys = _ffn(meta, xs, w1, w2)
    out = _combine(ys, pos)
    return out.reshape(b, s, h)
